# baseline (speedup 1.0000x reference)
import functools

import jax
import jax.numpy as jnp
from jax import lax
from jax.experimental import pallas as pl
from jax.experimental.pallas import tpu as pltpu
from jax.experimental.pallas import tpu_sc as plsc

N_NODES = 100000
N_EDGES = 1600000
NDIM = 32

NUM_CORES = 2
NUM_SUBCORES = 16
NW = NUM_CORES * NUM_SUBCORES
CHUNK = 2560
N_CHUNKS_TOT = N_EDGES // CHUNK
ITERS = -(-N_CHUNKS_TOT // NW)
GROUPS = CHUNK // 16
LANES = 16
PADN = 100352
SLICE = PADN // NUM_SUBCORES


def _sc_histogram(edge_index, off16):
    mesh = plsc.VectorSubcoreMesh(
        core_axis_name="c", subcore_axis_name="s",
        num_cores=NUM_CORES, num_subcores=NUM_SUBCORES)

    @functools.partial(
        pl.kernel,
        out_type=jax.ShapeDtypeStruct((NUM_CORES, PADN), jnp.float32),
        mesh=mesh,
        scratch_types=[
            pltpu.VMEM((PADN,), jnp.float32),
            pltpu.VMEM((2, CHUNK), jnp.int32),
            pltpu.VMEM((2, CHUNK), jnp.int32),
            pltpu.VMEM((SLICE,), jnp.float32),
            pltpu.VMEM((SLICE,), jnp.float32),
            pltpu.VMEM((LANES,), jnp.int32),
            pltpu.VMEM_SHARED((NUM_SUBCORES, SLICE), jnp.float32),
            pltpu.SemaphoreType.DMA,
            pltpu.SemaphoreType.DMA,
        ],
        compiler_params=pltpu.CompilerParams(needs_layout_passes=False),
    )
    def hist_kernel(edge_hbm, off_hbm, out_hbm, hist, buf0, buf1, tmp, acc,
                    off_v, shared, sem0, sem1):
        cid0 = lax.axis_index("c")
        sid = lax.axis_index("s")
        wid = sid * NUM_CORES + cid0

        pltpu.sync_copy(off_hbm, off_v)
        off = off_v[...]

        @plsc.parallel_loop(0, PADN // LANES, unroll=8)
        def _zero(i):
            hist[pl.ds(i * LANES, LANES)] = jnp.zeros((LANES,), jnp.float32)

        bufs = (buf0, buf1)
        sems = (sem0, sem1)

        def start(i):
            cid = jnp.minimum(i * NW + wid, N_CHUNKS_TOT - 1)
            base = pl.multiple_of(cid * CHUNK, 128)
            return pltpu.async_copy(
                edge_hbm.at[:, pl.ds(base, CHUNK)], bufs[i % 2], sems[i % 2])

        pending = start(0)
        for i in range(ITERS):
            nxt = start(i + 1) if i + 1 < ITERS else None
            pending.wait()
            buf = bufs[i % 2]
            valid = jnp.full((LANES,), i * NW + wid, jnp.int32) < N_CHUNKS_TOT

            @plsc.parallel_loop(0, GROUPS, unroll=8)
            def _scatter(j):
                idx = buf[1, pl.ds(j * LANES, LANES)] + off
                cnt, last = plsc.scan_count(idx)
                plsc.addupdate_scatter(hist, [idx], cnt.astype(jnp.float32),
                                       mask=last & valid)

            pending = nxt

        for r in range(NUM_SUBCORES):
            tgt = lax.rem(sid + r, NUM_SUBCORES)
            pltpu.sync_copy(hist.at[pl.ds(tgt * SLICE, SLICE)],
                            shared.at[sid])
            plsc.subcore_barrier()
            src = lax.rem(sid - r + NUM_SUBCORES, NUM_SUBCORES)
            pltpu.sync_copy(shared.at[src], tmp)
            if r == 0:
                @plsc.parallel_loop(0, SLICE // LANES, unroll=8)
                def _init(k):
                    sl = pl.ds(k * LANES, LANES)
                    acc[sl] = tmp[sl]
            else:
                @plsc.parallel_loop(0, SLICE // LANES, unroll=8)
                def _acc(k):
                    sl = pl.ds(k * LANES, LANES)
                    acc[sl] = acc[sl] + tmp[sl]
            plsc.subcore_barrier()

        pltpu.sync_copy(acc, out_hbm.at[cid0, pl.ds(sid * SLICE, SLICE)])

    return hist_kernel(edge_index, off16)


BN = 8192


def _tc_node_mlp(partial, W_e1, b_e1, W_e2, b_e2, W_n1, b_n1, W_n2, b_n2):
    grid = (pl.cdiv(N_NODES, BN),)

    def body(part_ref, we1, be1, we2, be2, wn1, bn1, wn2, bn2, out_ref):
        h = jnp.maximum(we1[...] + be1[...][None, :], 0.0)
        v = jnp.sum(h.reshape(NDIM, 1) * we2[...], axis=0)[None, :]
        v = v + be2[...][None, :]
        u = jnp.sum(v.reshape(NDIM, 1) * wn1[...], axis=0)[None, :]
        deg = jnp.sum(part_ref[...], axis=0)
        t = deg[:, None] * u + bn1[...][None, :]
        z = jnp.maximum(t, 0.0)
        o = jnp.dot(z, wn2[...], preferred_element_type=jnp.float32)
        out_ref[...] = o + bn2[...][None, :]

    full = lambda shape: pl.BlockSpec(shape, lambda i: tuple(0 for _ in shape))
    return pl.pallas_call(
        body,
        grid=grid,
        in_specs=[
            pl.BlockSpec((NUM_CORES, BN), lambda i: (0, i)),
            full((1, NDIM)), full((NDIM,)),
            full((NDIM, NDIM)), full((NDIM,)),
            full((NDIM, NDIM)), full((NDIM,)),
            full((NDIM, NDIM)), full((NDIM,)),
        ],
        out_specs=pl.BlockSpec((BN, NDIM), lambda i: (i, 0)),
        out_shape=jax.ShapeDtypeStruct((N_NODES, NDIM), jnp.float32),
    )(partial, W_e1, b_e1, W_e2, b_e2, W_n1, b_n1, W_n2, b_n2)


def kernel(edge_index, edge_weight, num_nodes, W_e1, b_e1, W_e2, b_e2,
           W_n1, b_n1, W_n2, b_n2):
    del edge_weight
    off = jnp.asarray(num_nodes, jnp.int32) - jnp.int32(N_NODES)
    off16 = jnp.full((LANES,), off, jnp.int32)
    partial = _sc_histogram(edge_index, off16)
    return _tc_node_mlp(partial, W_e1, b_e1, W_e2, b_e2, W_n1, b_n1, W_n2, b_n2)


# speedup vs baseline: 1.5832x; 1.5832x over previous
import functools

import jax
import jax.numpy as jnp
from jax import lax
from jax.experimental import pallas as pl
from jax.experimental.pallas import tpu as pltpu
from jax.experimental.pallas import tpu_sc as plsc

N_NODES = 100000
N_EDGES = 1600000
NDIM = 32

NUM_CORES = 2
NUM_SUBCORES = 16
NW = NUM_CORES * NUM_SUBCORES
CHUNK = 2560
N_CHUNKS_TOT = N_EDGES // CHUNK
ITERS = -(-N_CHUNKS_TOT // NW)
GROUPS = CHUNK // 16
LANES = 16
PADN = 100352
SLICE = PADN // NUM_SUBCORES


def _sc_histogram(edge_index, off16):
    mesh = plsc.VectorSubcoreMesh(
        core_axis_name="c", subcore_axis_name="s",
        num_cores=NUM_CORES, num_subcores=NUM_SUBCORES)

    @functools.partial(
        pl.kernel,
        out_type=jax.ShapeDtypeStruct((NUM_CORES, PADN), jnp.float32),
        mesh=mesh,
        scratch_types=[
            pltpu.VMEM((PADN,), jnp.float32),
            pltpu.VMEM((2, CHUNK), jnp.int32),
            pltpu.VMEM((2, CHUNK), jnp.int32),
            pltpu.VMEM((SLICE,), jnp.float32),
            pltpu.VMEM((SLICE,), jnp.float32),
            pltpu.VMEM((LANES,), jnp.int32),
            pltpu.VMEM_SHARED((NUM_SUBCORES, SLICE), jnp.float32),
            pltpu.SemaphoreType.DMA,
            pltpu.SemaphoreType.DMA,
        ],
        compiler_params=pltpu.CompilerParams(needs_layout_passes=False),
    )
    def hist_kernel(edge_hbm, off_hbm, out_hbm, hist, buf0, buf1, tmp, acc,
                    off_v, shared, sem0, sem1):
        cid0 = lax.axis_index("c")
        sid = lax.axis_index("s")
        wid = sid * NUM_CORES + cid0

        pltpu.sync_copy(off_hbm, off_v)
        off = off_v[...]

        @plsc.parallel_loop(0, PADN // LANES, unroll=8)
        def _zero(i):
            hist[pl.ds(i * LANES, LANES)] = jnp.zeros((LANES,), jnp.float32)

        bufs = (buf0, buf1)
        sems = (sem0, sem1)

        def start(i):
            cid = jnp.minimum(i * NW + wid, N_CHUNKS_TOT - 1)
            base = pl.multiple_of(cid * CHUNK, 128)
            return pltpu.async_copy(
                edge_hbm.at[:, pl.ds(base, CHUNK)], bufs[i % 2], sems[i % 2])

        pending = start(0)
        for i in range(ITERS):
            nxt = start(i + 1) if i + 1 < ITERS else None
            pending.wait()
            buf = bufs[i % 2]
            valid = jnp.full((LANES,), i * NW + wid, jnp.int32) < N_CHUNKS_TOT

            @plsc.parallel_loop(0, GROUPS, unroll=8)
            def _scatter(j):
                idx = buf[1, pl.ds(j * LANES, LANES)] + off
                cnt, last = plsc.scan_count(idx)
                plsc.addupdate_scatter(hist, [idx], cnt.astype(jnp.float32),
                                       mask=last & valid)

            pending = nxt

        for r in range(NUM_SUBCORES):
            tgt = lax.rem(sid + r, NUM_SUBCORES)
            pltpu.sync_copy(hist.at[pl.ds(tgt * SLICE, SLICE)],
                            shared.at[sid])
            plsc.subcore_barrier()
            src = lax.rem(sid - r + NUM_SUBCORES, NUM_SUBCORES)
            pltpu.sync_copy(shared.at[src], tmp)
            if r == 0:
                @plsc.parallel_loop(0, SLICE // LANES, unroll=8)
                def _init(k):
                    sl = pl.ds(k * LANES, LANES)
                    acc[sl] = tmp[sl]
            else:
                @plsc.parallel_loop(0, SLICE // LANES, unroll=8)
                def _acc(k):
                    sl = pl.ds(k * LANES, LANES)
                    acc[sl] = acc[sl] + tmp[sl]
            plsc.subcore_barrier()

        pltpu.sync_copy(acc, out_hbm.at[cid0, pl.ds(sid * SLICE, SLICE)])

    return hist_kernel(edge_index, off16)


BN = 8192


def _tc_node_mlp(partial, u_col, bn1_col, wn2_t, bn2_col):
    grid = (pl.cdiv(N_NODES, BN),)

    def body(part_ref, ucol, bn1c, wn2t, bn2c, out_ref):
        deg = jnp.sum(part_ref[...], axis=0)
        z = jnp.maximum(ucol[...] * deg[None, :] + bn1c[...], 0.0)
        o = jnp.dot(wn2t[...], z, preferred_element_type=jnp.float32)
        out_ref[...] = o + bn2c[...]

    full = lambda shape: pl.BlockSpec(shape, lambda i: tuple(0 for _ in shape))
    return pl.pallas_call(
        body,
        grid=grid,
        in_specs=[
            pl.BlockSpec((NUM_CORES, BN), lambda i: (0, i)),
            full((NDIM, 1)), full((NDIM, 1)),
            full((NDIM, NDIM)), full((NDIM, 1)),
        ],
        out_specs=pl.BlockSpec((NDIM, BN), lambda i: (0, i)),
        out_shape=jax.ShapeDtypeStruct((NDIM, N_NODES), jnp.float32),
    )(partial, u_col, bn1_col, wn2_t, bn2_col)


def kernel(edge_index, edge_weight, num_nodes, W_e1, b_e1, W_e2, b_e2,
           W_n1, b_n1, W_n2, b_n2):
    del edge_weight
    off = jnp.asarray(num_nodes, jnp.int32) - jnp.int32(N_NODES)
    off16 = jnp.full((LANES,), off, jnp.int32)
    partial = _sc_histogram(edge_index, off16)
    h = jax.nn.relu(W_e1[0] + b_e1)
    v = h @ W_e2 + b_e2
    u = v @ W_n1
    out_t = _tc_node_mlp(partial, u[:, None], b_n1[:, None],
                         W_n2.T, b_n2[:, None])
    return out_t.T
